# Initial kernel scaffold; baseline (speedup 1.0000x reference)
import jax
import jax.numpy as jnp
from jax.experimental import pallas as pl


def kernel(t, pos, poi_t, poi_pos, batch, W0, b0, W1, b1, W2, b2, W3, b3, W4, b4):
    raise NotImplementedError("write your pallas kernel here")


# speedup vs baseline: 5.0535x; 5.0535x over previous
import functools

import jax
import jax.numpy as jnp
from jax import lax
from jax.experimental import pallas as pl
from jax.experimental.pallas import tpu as pltpu
from jax.experimental.pallas import tpu_sc as plsc

N = 1_600_000
B = 100_000
Np = 1_638_400
Bp = 102_400
ROWS2D = Np // 128
NW = 32
RPT = Np // NW
KCH = 6400
NCH = RPT // KCH
SEG_PT = Bp // 16


def _gather_body(tab_hbm, batch_hbm, g_hbm, idx_v, rows_v, sem):
    cid = lax.axis_index("c")
    sid = lax.axis_index("s")
    wid = sid * 2 + cid
    for k in range(NCH):
        base = wid * RPT + k * KCH
        pltpu.sync_copy(batch_hbm.at[pl.ds(base, KCH)], idx_v)
        pltpu.async_copy(tab_hbm.at[idx_v], rows_v, sem).wait()
        pltpu.sync_copy(rows_v, g_hbm.at[pl.ds(base, KCH)])


_gather_call = pl.kernel(
    _gather_body,
    out_type=jax.ShapeDtypeStruct((Np, 8), jnp.float32),
    mesh=plsc.VectorSubcoreMesh(core_axis_name="c", subcore_axis_name="s"),
    compiler_params=pltpu.CompilerParams(use_tc_tiling_on_sc=False),
    scratch_types=[
        pltpu.VMEM((KCH,), jnp.int32),
        pltpu.VMEM((KCH, 8), jnp.float32),
        pltpu.SemaphoreType.DMA,
    ],
)


def _mlp_body(t_ref, pos_ref, g_ref, W0, b0, W1, b1, W2, b2, W3, b3, W4, b4,
              cx_ref, cy_ref, cz_ref):
    tb = t_ref[...]
    px, py, pz = pos_ref[0], pos_ref[1], pos_ref[2]
    gt, gx, gy, gz = g_ref[0], g_ref[1], g_ref[2], g_ref[3]
    s = jnp.sign(tb - gt)
    dx = px - gx
    dy = py - gy
    dz = pz - gz
    r2 = dx * dx + dy * dy + dz * dz

    h = [s, r2]
    for W, b, fin, fout, relu in (
        (W0, b0, 2, 10, True),
        (W1, b1, 10, 20, True),
        (W2, b2, 20, 10, True),
        (W3, b3, 10, 5, True),
        (W4, b4, 5, 1, False),
    ):
        nxt = []
        for j in range(fout):
            acc = h[0] * W[j, 0] + b[j]
            for k in range(1, fin):
                acc = acc + h[k] * W[j, k]
            nxt.append(jnp.maximum(acc, 0.0) if relu else acc)
        h = nxt

    f = h[0] * lax.rsqrt(jnp.maximum(r2, 1e-24))
    cx_ref[...] = f * dx
    cy_ref[...] = f * dy
    cz_ref[...] = f * dz


def _mlp_call(t2, posT, g4, *wb):
    BR = 512
    grid = (ROWS2D // BR,)
    smem = pl.BlockSpec(memory_space=pltpu.MemorySpace.SMEM)
    return pl.pallas_call(
        _mlp_body,
        grid=grid,
        in_specs=[
            pl.BlockSpec((BR, 128), lambda i: (i, 0)),
            pl.BlockSpec((3, BR, 128), lambda i: (0, i, 0)),
            pl.BlockSpec((4, BR, 128), lambda i: (0, i, 0)),
        ] + [smem] * 10,
        out_specs=[pl.BlockSpec((BR, 128), lambda i: (i, 0))] * 3,
        out_shape=[jax.ShapeDtypeStruct((ROWS2D, 128), jnp.float32)] * 3,
    )(t2, posT, g4, *wb)


WIN = Bp // NW
Npp = Np + KCH


def _scatter_body(c_hbm, b_hbm, bounds_hbm, out_hbm, bounds_v, idx_v, c_v, acc):
    cid = lax.axis_index("c")
    sid = lax.axis_index("s")
    wid = sid * 2 + cid
    pltpu.sync_copy(bounds_hbm, bounds_v)
    bv = bounds_v[pl.ds(wid, 16)]
    r_lo = bv[0]
    r_hi = bv[1]
    start = (r_lo // 8) * 8
    nch = (r_hi - start + KCH - 1) // KCH
    wbase = wid * WIN

    def zero(i, _):
        acc[pl.ds(i * 16, 16)] = jnp.zeros((16,), jnp.float32)
        return 0

    lax.fori_loop(0, WIN * 4 // 16, zero, 0)

    lane = lax.iota(jnp.int32, 16)

    def chunk(k, _):
        off = start + k * KCH
        pltpu.sync_copy(b_hbm.at[pl.ds(off, KCH)], idx_v)
        pltpu.sync_copy(c_hbm.at[pl.ds(off, KCH)], c_v)

        def grp(i, _):
            rows16 = lane + i * 16
            ids = idx_v[pl.ds(i * 16, 16)]
            rowpos = rows16 + off
            valid = (rowpos >= r_lo) & (rowpos < r_hi)
            local = jnp.where(valid, ids - wbase, 0)
            flat = local * 4
            for c in range(3):
                vals = plsc.load_gather(
                    c_v, [rows16, jnp.full((16,), c, jnp.int32)])
                plsc.addupdate_scatter(acc, [flat + c], vals, mask=valid)
            return 0

        lax.fori_loop(0, KCH // 16, grp, 0)
        return 0

    lax.fori_loop(0, nch, chunk, 0)
    pltpu.sync_copy(acc, out_hbm.at[pl.ds(wid * WIN * 4, WIN * 4)])


_scatter_call = pl.kernel(
    _scatter_body,
    out_type=jax.ShapeDtypeStruct((Bp * 4,), jnp.float32),
    mesh=plsc.VectorSubcoreMesh(core_axis_name="c", subcore_axis_name="s"),
    compiler_params=pltpu.CompilerParams(use_tc_tiling_on_sc=False,
                                         needs_layout_passes=False),
    scratch_types=[
        pltpu.VMEM((48,), jnp.int32),
        pltpu.VMEM((KCH,), jnp.int32),
        pltpu.VMEM((KCH, 4), jnp.float32),
        pltpu.VMEM((WIN * 4,), jnp.float32),
    ],
)


def _norm_body(a_ref, o_ref):
    p = a_ref[...]
    sq = p * p
    r = lax.broadcasted_iota(jnp.int32, (128, 128), 0)
    c = lax.broadcasted_iota(jnp.int32, (128, 128), 1)
    M = ((r // 4) == (c // 4)).astype(jnp.float32)
    n2 = lax.dot_general(sq, M, (((1,), (0,)), ((), ())),
                         preferred_element_type=jnp.float32)
    n = jnp.sqrt(jnp.maximum(n2, 1e-24))
    o_ref[...] = p / n


def _norm_call(a):
    BRn = 400
    rows = (Bp * 4) // 128
    return pl.pallas_call(
        _norm_body,
        grid=(rows // BRn,),
        in_specs=[pl.BlockSpec((BRn, 128), lambda i: (i, 0))],
        out_specs=pl.BlockSpec((BRn, 128), lambda i: (i, 0)),
        out_shape=jax.ShapeDtypeStruct((rows, 128), jnp.float32),
    )(a)


def kernel(t, pos, poi_t, poi_pos, batch, W0, b0, W1, b1, W2, b2, W3, b3, W4, b4):
    f32 = jnp.float32
    tab = jnp.concatenate(
        [poi_t[:, None], poi_pos, jnp.zeros((B, 4), f32)], axis=1)
    tab = jnp.pad(tab, ((0, Bp - B), (0, 0)))

    batch_p = jnp.concatenate(
        [batch, jnp.full((Np - N,), Bp - 1, jnp.int32)])

    g = _gather_call(tab, batch_p)

    t2 = jnp.pad(t, (0, Np - N)).reshape(ROWS2D, 128)
    posT = jnp.pad(pos, ((0, Np - N), (0, 0))).T.reshape(3, ROWS2D, 128)
    g4 = g[:, :4].T.reshape(4, ROWS2D, 128)

    cx, cy, cz = _mlp_call(t2, posT, g4,
                           W0, b0, W1, b1, W2, b2, W3, b3, W4, b4)

    c4 = jnp.stack([cx, cy, cz, jnp.zeros_like(cx)], axis=-1)
    c4r = jnp.pad(c4.reshape(Np, 4), ((0, Npp - Np), (0, 0)))
    batch_pp = jnp.concatenate(
        [batch_p, jnp.full((Npp - Np,), Bp - 1, jnp.int32)])
    bounds = jnp.searchsorted(
        batch_pp[:Np], jnp.arange(33, dtype=jnp.int32) * WIN).astype(jnp.int32)
    bounds = jnp.pad(bounds, (0, 15))

    acc = _scatter_call(c4r, batch_pp, bounds)

    o = _norm_call(acc.reshape((Bp * 4) // 128, 128))
    return o.reshape(Bp, 4)[:B, :3]
